# Initial kernel scaffold; baseline (speedup 1.0000x reference)
import jax
import jax.numpy as jnp
from jax.experimental import pallas as pl


def kernel(x):
    raise NotImplementedError("write your pallas kernel here")


# speedup vs baseline: 1.4716x; 1.4716x over previous
import math

import jax
import jax.numpy as jnp
from jax import lax
from jax.experimental import pallas as pl
from jax.experimental.pallas import tpu as pltpu
from jax.experimental.pallas import tpu_sc as plsc

EMBED = 1024
HALF = EMBED // 2
PAD = 1
BATCH = 4
SEQ = 8192
SCALE = math.log(10000.0) / (HALF - 1)

V_PAD = 8704
TROWS = 512
NC, NS, L = 2, 16, 16
NW = NC * NS
FLAT = BATCH * SEQ
RW = FLAT // NW
CH = 32
NCHUNK = RW // CH


def _table_body(o_ref):
    t = pl.program_id(0)
    row = lax.broadcasted_iota(jnp.int32, (TROWS, EMBED), 0) + t * TROWS
    col = lax.broadcasted_iota(jnp.int32, (TROWS, EMBED), 1)
    k = jnp.where(col < HALF, col, col - HALF).astype(jnp.float32)
    inv_freq = jnp.exp(k * (-SCALE))
    ang = row.astype(jnp.float32) * inv_freq
    val = jnp.where(col < HALF, jnp.sin(ang), jnp.cos(ang))
    o_ref[...] = jnp.where(row == PAD, 0.0, val)


def _build_table():
    return pl.pallas_call(
        _table_body,
        out_shape=jax.ShapeDtypeStruct((V_PAD, EMBED), jnp.float32),
        grid=(V_PAD // TROWS,),
        out_specs=pl.BlockSpec((TROWS, EMBED), lambda t: (t, 0)),
    )()


def _sc_body(x_hbm, tab_hbm, out_hbm, x_v, idx_v, buf_v, gsem):
    wid = lax.axis_index("s") * NC + lax.axis_index("c")
    base = wid * RW
    s_base = (wid % (SEQ // RW)) * RW
    pltpu.sync_copy(x_hbm.at[pl.ds(base, RW)], x_v)
    iota = lax.iota(jnp.int32, L)

    @pl.loop(0, RW // L)
    def _(i):
        xv = x_v[pl.ds(i * L, L)]
        pos = s_base + i * L + (PAD + 1) + iota
        idx_v[pl.ds(i * L, L)] = jnp.where(xv == PAD, PAD, pos)

    @pl.loop(0, NCHUNK)
    def _(g):
        pltpu.async_copy(
            tab_hbm.at[idx_v.at[pl.ds(g * CH, CH)]], buf_v, gsem
        ).wait()
        pltpu.sync_copy(buf_v, out_hbm.at[pl.ds(base + g * CH, CH)])


def kernel(x):
    tab = _build_table()
    mesh = plsc.VectorSubcoreMesh(core_axis_name="c", subcore_axis_name="s")
    sck = pl.kernel(
        _sc_body,
        out_type=jax.ShapeDtypeStruct((FLAT, EMBED), jnp.float32),
        mesh=mesh,
        scratch_types=[
            pltpu.VMEM((RW,), jnp.int32),
            pltpu.VMEM((RW,), jnp.int32),
            pltpu.VMEM((CH, EMBED), jnp.float32),
            pltpu.SemaphoreType.DMA,
        ],
    )
    out = sck(x.reshape(FLAT), tab)
    return out.reshape(BATCH, SEQ, EMBED)


# speedup vs baseline: 1.6455x; 1.1182x over previous
import math

import jax
import jax.numpy as jnp
from jax import lax
from jax.experimental import pallas as pl
from jax.experimental.pallas import tpu as pltpu
from jax.experimental.pallas import tpu_sc as plsc

EMBED = 1024
HALF = EMBED // 2
PAD = 1
BATCH = 4
SEQ = 8192
SCALE = math.log(10000.0) / (HALF - 1)

V_PAD = 8704
TROWS = 512
NC, NS, L = 2, 16, 16
NW = NC * NS
FLAT = BATCH * SEQ
RW = FLAT // NW
CH = 16
NCHUNK = RW // CH
NBUF = 4


def _table_body(o_ref):
    t = pl.program_id(0)
    row = lax.broadcasted_iota(jnp.int32, (TROWS, EMBED), 0) + t * TROWS
    col = lax.broadcasted_iota(jnp.int32, (TROWS, EMBED), 1)
    k = jnp.where(col < HALF, col, col - HALF).astype(jnp.float32)
    inv_freq = jnp.exp(k * (-SCALE))
    ang = row.astype(jnp.float32) * inv_freq
    val = jnp.where(col < HALF, jnp.sin(ang), jnp.cos(ang))
    o_ref[...] = jnp.where(row == PAD, 0.0, val)


def _build_table():
    return pl.pallas_call(
        _table_body,
        out_shape=jax.ShapeDtypeStruct((V_PAD, EMBED), jnp.float32),
        grid=(V_PAD // TROWS,),
        out_specs=pl.BlockSpec((TROWS, EMBED), lambda t: (t, 0)),
    )()


def _sc_body(x_hbm, tab_hbm, out_hbm, x_v, idx_v, buf_v, *sems):
    gsems, psems = sems[:NBUF], sems[NBUF:]
    wid = lax.axis_index("s") * NC + lax.axis_index("c")
    base = wid * RW
    s_base = (wid % (SEQ // RW)) * RW
    pltpu.sync_copy(x_hbm.at[pl.ds(base, RW)], x_v)
    iota = lax.iota(jnp.int32, L)

    @pl.loop(0, RW // L)
    def _(i):
        xv = x_v[pl.ds(i * L, L)]
        pos = s_base + i * L + (PAD + 1) + iota
        idx_v[pl.ds(i * L, L)] = jnp.where(xv == PAD, PAD, pos)

    def start_gather(g, b):
        pltpu.async_copy(
            tab_hbm.at[idx_v.at[pl.ds(g * CH, CH)]], buf_v.at[b], gsems[b]
        )

    def wait_gather(b):
        pltpu.make_async_copy(
            tab_hbm.at[idx_v.at[pl.ds(0, CH)]], buf_v.at[b], gsems[b]
        ).wait()

    def start_put(g, b):
        pltpu.async_copy(
            buf_v.at[b], out_hbm.at[pl.ds(base + g * CH, CH)], psems[b]
        )

    def wait_put(b):
        pltpu.make_async_copy(
            buf_v.at[b], out_hbm.at[pl.ds(base, CH)], psems[b]
        ).wait()

    start_gather(0, 0)
    start_gather(1, 1)
    for g in (0, 1):
        wait_gather(g % NBUF)
        start_put(g, g % NBUF)
        start_gather(g + 2, (g + 2) % NBUF)

    @pl.loop(2, NCHUNK - 2, step=NBUF)
    def _(g0):
        for j in range(NBUF):
            g = g0 + j
            b = (2 + j) % NBUF
            bn = (j + 4) % NBUF
            wait_put(bn)
            start_gather(g + 2, bn)
            wait_gather(b)
            start_put(g, b)

    for g in (NCHUNK - 2, NCHUNK - 1):
        wait_gather(g % NBUF)
        start_put(g, g % NBUF)
    for b in range(NBUF):
        wait_put(b)


def kernel(x):
    tab = _build_table()
    mesh = plsc.VectorSubcoreMesh(core_axis_name="c", subcore_axis_name="s")
    sck = pl.kernel(
        _sc_body,
        out_type=jax.ShapeDtypeStruct((FLAT, EMBED), jnp.float32),
        mesh=mesh,
        scratch_types=[
            pltpu.VMEM((RW,), jnp.int32),
            pltpu.VMEM((RW,), jnp.int32),
            pltpu.VMEM((NBUF, CH, EMBED), jnp.float32),
        ] + [pltpu.SemaphoreType.DMA] * (2 * NBUF),
    )
    out = sck(x.reshape(FLAT), tab)
    return out.reshape(BATCH, SEQ, EMBED)


# speedup vs baseline: 2.6373x; 1.6027x over previous
import math

import jax
import jax.numpy as jnp
import numpy as np
from jax import lax
from jax.experimental import pallas as pl
from jax.experimental.pallas import tpu as pltpu
from jax.experimental.pallas import tpu_sc as plsc

EMBED = 1024
HALF = EMBED // 2
PAD = 1
BATCH = 4
SEQ = 8192
SCALE = math.log(10000.0) / (HALF - 1)

V_PAD = 8704
ZBASE = 8200
NC, NS, L = 2, 16, 16
NW = NC * NS
FLAT = BATCH * SEQ
RW = FLAT // NW
CH = 16
NCHUNK = RW // CH
NBUF = 4
NPEER = SEQ // RW


def _make_table() -> np.ndarray:
    freqs = np.exp(np.arange(HALF, dtype=np.float32) * np.float32(-SCALE))
    pos = np.arange(SEQ + 2, dtype=np.float32)
    ang = pos[:, None] * freqs[None, :]
    tab = np.zeros((V_PAD, EMBED), dtype=np.float32)
    tab[: SEQ + 2, :HALF] = np.sin(ang)
    tab[: SEQ + 2, HALF:] = np.cos(ang)
    tab[PAD, :] = 0.0
    return tab


_TABLE = _make_table()


def _sc_body(x_hbm, tab_hbm, out_hbm, x_v, idx_v, buf_v, *sems):
    gsems, psems = sems[:NBUF], sems[NBUF:]
    wid = lax.axis_index("s") * NC + lax.axis_index("c")
    base = wid * RW
    s_base = (wid % NPEER) * RW
    stag = (wid // NPEER) * (NCHUNK // BATCH)
    pltpu.sync_copy(x_hbm.at[pl.ds(base, RW)], x_v)
    iota = lax.iota(jnp.int32, L)

    @pl.loop(0, RW // L)
    def _(i):
        xv = x_v[pl.ds(i * L, L)]
        loc = i * L + iota
        pos = s_base + (PAD + 1) + loc
        zrow = ZBASE + (loc & 255)
        idx_v[pl.ds(i * L, L)] = jnp.where(xv == PAD, zrow, pos)

    def chunk(g):
        return lax.rem(g + stag, NCHUNK)

    def start_gather(g, b):
        c = chunk(g)
        pltpu.async_copy(
            tab_hbm.at[idx_v.at[pl.ds(c * CH, CH)]], buf_v.at[b], gsems[b]
        )

    def wait_gather(b):
        pltpu.make_async_copy(
            tab_hbm.at[idx_v.at[pl.ds(0, CH)]], buf_v.at[b], gsems[b]
        ).wait()

    def start_put(g, b):
        c = chunk(g)
        pltpu.async_copy(
            buf_v.at[b], out_hbm.at[pl.ds(base + c * CH, CH)], psems[b]
        )

    def wait_put(b):
        pltpu.make_async_copy(
            buf_v.at[b], out_hbm.at[pl.ds(base, CH)], psems[b]
        ).wait()

    start_gather(0, 0)
    start_gather(1, 1)
    for g in (0, 1):
        wait_gather(g % NBUF)
        start_put(g, g % NBUF)
        start_gather(g + 2, (g + 2) % NBUF)

    @pl.loop(2, NCHUNK - 2, step=NBUF)
    def _(g0):
        for j in range(NBUF):
            g = g0 + j
            b = (2 + j) % NBUF
            bn = j
            wait_put(bn)
            start_gather(g + 2, bn)
            wait_gather(b)
            start_put(g, b)

    for g in (NCHUNK - 2, NCHUNK - 1):
        wait_gather(g % NBUF)
        start_put(g, g % NBUF)
    for b in range(NBUF):
        wait_put(b)


def kernel(x):
    tab = jnp.asarray(_TABLE)
    mesh = plsc.VectorSubcoreMesh(core_axis_name="c", subcore_axis_name="s")
    sck = pl.kernel(
        _sc_body,
        out_type=jax.ShapeDtypeStruct((FLAT, EMBED), jnp.float32),
        mesh=mesh,
        scratch_types=[
            pltpu.VMEM((RW,), jnp.int32),
            pltpu.VMEM((RW,), jnp.int32),
            pltpu.VMEM((NBUF, CH, EMBED), jnp.float32),
        ] + [pltpu.SemaphoreType.DMA] * (2 * NBUF),
    )
    out = sck(x.reshape(FLAT), tab)
    return out.reshape(BATCH, SEQ, EMBED)


# speedup vs baseline: 3.2517x; 1.2330x over previous
import math

import jax
import jax.numpy as jnp
import numpy as np
from jax import lax
from jax.experimental import pallas as pl
from jax.experimental.pallas import tpu as pltpu
from jax.experimental.pallas import tpu_sc as plsc

EMBED = 1024
HALF = EMBED // 2
PAD = 1
BATCH = 4
SEQ = 8192
SCALE = math.log(10000.0) / (HALF - 1)

V_PAD = 8704
ZROWS = 8448
NC, NS, L = 2, 16, 16
NW = NC * NS
FLAT = BATCH * SEQ
SW = SEQ // NW
CH = 32
NCH = SW // CH
NBUF = 3


def _make_table() -> np.ndarray:
    freqs = np.exp(np.arange(HALF, dtype=np.float32) * np.float32(-SCALE))
    pos = np.arange(2, SEQ + 2, dtype=np.float32)
    ang = pos[:, None] * freqs[None, :]
    tab = np.zeros((V_PAD, EMBED), dtype=np.float32)
    tab[:SEQ, :HALF] = np.sin(ang)
    tab[:SEQ, HALF:] = np.cos(ang)
    return tab


_TABLE = _make_table()


def _sc_body(x_hbm, tab_hbm, out_hbm, x_v, buf_v, zbuf_v, zidx_v, *sems):
    gsems, psems = sems[:NBUF], sems[NBUF:NBUF + NBUF]
    fsem = sems[2 * NBUF]
    wid = lax.axis_index("s") * NC + lax.axis_index("c")
    s_base = wid * SW
    for bb in range(BATCH):
        pltpu.sync_copy(x_hbm.at[pl.ds(bb * SEQ + s_base, SW)],
                        x_v.at[bb])
    pltpu.sync_copy(tab_hbm.at[pl.ds(ZROWS, L)], zbuf_v)

    def start_gather(c, b):
        pltpu.async_copy(
            tab_hbm.at[pl.ds(s_base + c * CH, CH)], buf_v.at[b], gsems[b]
        )

    def wait_gather(b):
        pltpu.make_async_copy(
            tab_hbm.at[pl.ds(0, CH)], buf_v.at[b], gsems[b]
        ).wait()

    def start_puts(c, b):
        for bb in range(BATCH):
            pltpu.async_copy(
                buf_v.at[b],
                out_hbm.at[pl.ds(bb * SEQ + s_base + c * CH, CH)],
                psems[b],
            )

    def wait_puts(b):
        for _ in range(BATCH):
            pltpu.make_async_copy(
                buf_v.at[b], out_hbm.at[pl.ds(0, CH)], psems[b]
            ).wait()

    for c in range(NBUF):
        start_gather(c, c)
    for c in range(NCH):
        b = c % NBUF
        if c >= 1:
            bn = (c + 2) % NBUF
            wait_puts(bn)
            if c + 2 < NCH:
                start_gather(c + 2, bn)
        wait_gather(b)
        start_puts(c, b)
    wait_puts((NCH - 1) % NBUF)

    iota = lax.iota(jnp.int32, L)
    for bb in range(BATCH):
        out_base = bb * SEQ + s_base

        @pl.loop(0, SW // L)
        def _(v):
            xv = x_v[bb, pl.ds(v * L, L)]
            pm = xv == PAD
            npad = jnp.sum(jnp.where(pm, 1, 0))

            @pl.when(npad > 0)
            def _():
                rows = out_base + v * L + iota
                first = out_base + v * L + plsc.all_reduce_ffs(pm)
                zidx_v[0, :] = jnp.where(pm, rows, first)
                pltpu.async_copy(zbuf_v, out_hbm.at[zidx_v.at[0]], fsem).wait()


def kernel(x):
    tab = jnp.asarray(_TABLE)
    mesh = plsc.VectorSubcoreMesh(core_axis_name="c", subcore_axis_name="s")
    sck = pl.kernel(
        _sc_body,
        out_type=jax.ShapeDtypeStruct((FLAT, EMBED), jnp.float32),
        mesh=mesh,
        scratch_types=[
            pltpu.VMEM((BATCH, SW), jnp.int32),
            pltpu.VMEM((NBUF, CH, EMBED), jnp.float32),
            pltpu.VMEM((L, EMBED), jnp.float32),
            pltpu.VMEM((1, L), jnp.int32),
        ] + [pltpu.SemaphoreType.DMA] * (2 * NBUF + 1),
        compiler_params=pltpu.CompilerParams(needs_layout_passes=False),
    )
    out = sck(x.reshape(FLAT), tab)
    return out.reshape(BATCH, SEQ, EMBED)
